# baseline (speedup 1.0000x reference)
import functools

import jax
import jax.numpy as jnp
from jax import lax
from jax.experimental import pallas as pl
from jax.experimental.pallas import tpu as pltpu
from jax.experimental.pallas import tpu_sc as plsc

N = 10000
D = 128
E = 320000
NP = 10240
NW = 32
K = 88
NBLK = 120
EPW = K * NBLK
EP = NW * EPW
RPT = NP // 16
NBUF = 4
NSLOT = 8

def _mesh():
    return plsc.VectorSubcoreMesh(
        core_axis_name="c", subcore_axis_name="s", num_cores=2,
        num_subcores=16)


def _deg_body(dst_hbm, zeros1_hbm, ones_hbm, out_hbm, didx, ones_v, deg_sp):
    c = lax.axis_index("c")
    s = lax.axis_index("s")
    w = s * 2 + c
    pltpu.sync_copy(dst_hbm.at[w], didx)
    pltpu.sync_copy(ones_hbm, ones_v)
    pltpu.sync_copy(zeros1_hbm, deg_sp.at[pl.ds(s * RPT, RPT)])
    plsc.subcore_barrier()

    def body(j, carry):
        pltpu.sync_copy(ones_v, deg_sp.at[didx.at[j]], add=True)
        return carry

    lax.fori_loop(0, NBLK, body, 0)
    plsc.subcore_barrier()
    pltpu.sync_copy(deg_sp.at[pl.ds(s * RPT, RPT)],
                    out_hbm.at[c, pl.ds(s * RPT, RPT)])


@functools.lru_cache(maxsize=None)
def _deg_kernel():
    return pl.kernel(
        _deg_body,
        out_type=jax.ShapeDtypeStruct((2, NP), jnp.float32),
        mesh=_mesh(),
        scratch_types=[
            pltpu.VMEM((NBLK, K), jnp.int32),
            pltpu.VMEM((K,), jnp.float32),
            pltpu.VMEM_SHARED((NP,), jnp.float32),
        ],
    )


def _edge_body(y_hbm, sd_hbm, zrows_hbm, out_hbm,
               ring, bufs, acc_sp,
               isem0, isem1, isem2, isem3, isem4, isem5, isem6, isem7,
               gsem0, gsem1, gsem2, gsem3, ssem0, ssem1, ssem2, ssem3):
    c = lax.axis_index("c")
    s = lax.axis_index("s")
    w = s * 2 + c
    isems = (isem0, isem1, isem2, isem3, isem4, isem5, isem6, isem7)
    gsems = (gsem0, gsem1, gsem2, gsem3)
    ssems = (ssem0, ssem1, ssem2, ssem3)

    def iload(blk, slot):
        return pltpu.async_copy(
            sd_hbm.at[w, blk], ring.at[pl.ds(2 * slot, 2)], isems[slot])

    def iwait(slot):
        pltpu.make_async_copy(sd_hbm.at[w, 0], ring.at[pl.ds(2 * slot, 2)],
                              isems[slot]).wait()

    def gather(blk, slot, buf):
        return pltpu.async_copy(
            y_hbm.at[ring.at[2 * slot]], bufs.at[buf], gsems[buf])

    def gwait(slot, buf):
        pltpu.make_async_copy(y_hbm.at[ring.at[2 * slot]], bufs.at[buf],
                              gsems[buf]).wait()

    def scat(slot, buf):
        return pltpu.async_copy(
            bufs.at[buf], acc_sp.at[ring.at[2 * slot + 1]], ssems[buf],
            add=True)

    def swait(slot, buf):
        pltpu.make_async_copy(bufs.at[buf], acc_sp.at[ring.at[2 * slot + 1]],
                              ssems[buf]).wait()

    pltpu.sync_copy(zrows_hbm, acc_sp.at[pl.ds(s * RPT, RPT)])
    plsc.subcore_barrier()

    for r in range(6):
        iload(r, r)
    for r in range(3):
        iwait(r)
        gather(r, r, r)

    def body(j, carry):
        for b in range(8):
            jb = j * 8 + b
            sl = b % NSLOT
            bf = b % NBUF
            gwait(sl, bf)
            scat(sl, bf)

            @pl.when(jb + 3 < NBLK)
            def _next_gather():
                @pl.when(jb >= 1)
                def _wait_prev_scat():
                    swait((sl - 1) % NSLOT, (bf - 1) % NBUF)
                iwait((sl + 3) % NSLOT)
                gather(jb + 3, (sl + 3) % NSLOT, (bf + 3) % NBUF)

                @pl.when(jb >= 1)
                def _next_iload():
                    @pl.when(jb + 5 < NBLK)
                    def _do_iload():
                        iload(jb + 5, (sl + 5) % NSLOT)
        return carry

    lax.fori_loop(0, NBLK // 8, body, 0)
    for t in range(4):
        blk = NBLK - 4 + t
        swait(blk % NSLOT, blk % NBUF)
    plsc.subcore_barrier()
    pltpu.sync_copy(acc_sp.at[pl.ds(s * RPT, RPT)],
                    out_hbm.at[c, pl.ds(s * RPT, RPT)])


@functools.lru_cache(maxsize=None)
def _edge_kernel():
    return pl.kernel(
        _edge_body,
        out_type=jax.ShapeDtypeStruct((2, NP, D), jnp.float32),
        mesh=_mesh(),
        scratch_types=[
            pltpu.VMEM((2 * NSLOT, K), jnp.int32),
            pltpu.VMEM((NBUF, K, D), jnp.float32),
            pltpu.VMEM_SHARED((NP, D), jnp.float32),
        ] + [pltpu.SemaphoreType.DMA] * 16,
    )


_BM = 1024
_GRID = NP // _BM


def _dinv_col(deg_ref):
    deg = deg_ref[...]
    ones = jnp.ones((2, 1), jnp.float32)
    degc = lax.dot_general(deg, ones, (((0,), (0,)), ((), ())))
    return 1.0 / jnp.sqrt(degc + 1.0)


def _mm_scale_body(deg_ref, x_ref, w_ref, o_ref):
    dinv = _dinv_col(deg_ref)
    xw = jnp.dot(x_ref[...], w_ref[...], preferred_element_type=jnp.float32)
    o_ref[...] = xw * dinv


_mm_scale = pl.pallas_call(
    _mm_scale_body,
    grid=(_GRID,),
    in_specs=[
        pl.BlockSpec((2, _BM), lambda i: (0, i)),
        pl.BlockSpec((_BM, D), lambda i: (i, 0)),
        pl.BlockSpec((D, D), lambda i: (0, 0)),
    ],
    out_specs=pl.BlockSpec((_BM, D), lambda i: (i, 0)),
    out_shape=jax.ShapeDtypeStruct((NP, D), jnp.float32),
)


def _mid_body(deg_ref, a_ref, y_ref, b_ref, w_ref, o_ref):
    dinv = _dinv_col(deg_ref)
    h = (a_ref[0] + a_ref[1] + y_ref[...]) * dinv + b_ref[...]
    o_ref[...] = jnp.dot(h, w_ref[...],
                         preferred_element_type=jnp.float32) * dinv


_mid = pl.pallas_call(
    _mid_body,
    grid=(_GRID,),
    in_specs=[
        pl.BlockSpec((2, _BM), lambda i: (0, i)),
        pl.BlockSpec((2, _BM, D), lambda i: (0, i, 0)),
        pl.BlockSpec((_BM, D), lambda i: (i, 0)),
        pl.BlockSpec((1, D), lambda i: (0, 0)),
        pl.BlockSpec((D, D), lambda i: (0, 0)),
    ],
    out_specs=pl.BlockSpec((_BM, D), lambda i: (i, 0)),
    out_shape=jax.ShapeDtypeStruct((NP, D), jnp.float32),
)


def _fin_body(deg_ref, a_ref, y_ref, b_ref, o_ref):
    dinv = _dinv_col(deg_ref)
    o_ref[...] = (a_ref[0] + a_ref[1] + y_ref[...]) * dinv + b_ref[...]


_fin = pl.pallas_call(
    _fin_body,
    grid=(_GRID,),
    in_specs=[
        pl.BlockSpec((2, _BM), lambda i: (0, i)),
        pl.BlockSpec((2, _BM, D), lambda i: (0, i, 0)),
        pl.BlockSpec((_BM, D), lambda i: (i, 0)),
        pl.BlockSpec((1, D), lambda i: (0, 0)),
    ],
    out_specs=pl.BlockSpec((_BM, D), lambda i: (i, 0)),
    out_shape=jax.ShapeDtypeStruct((NP, D), jnp.float32),
)


def kernel(x, edge_index, W1, b1, W2, b2):
    src, dst = edge_index[0], edge_index[1]
    npad = EP - E
    pad_src = jnp.arange(npad, dtype=jnp.int32) % N
    pad_dst = N + (jnp.arange(npad, dtype=jnp.int32) % (NP - N))
    src_p = jnp.concatenate([src, pad_src]).reshape(NW, NBLK, K)
    dst_p = jnp.concatenate([dst, pad_dst]).reshape(NW, NBLK, K)
    sd_p = jnp.stack([src_p, dst_p], axis=2)

    x_p = jnp.pad(x, ((0, NP - N), (0, 0)))
    b1r = b1.reshape(1, D)
    b2r = b2.reshape(1, D)

    zeros1 = jnp.zeros((RPT,), jnp.float32)
    ones_k = jnp.ones((K,), jnp.float32)
    zrows = jnp.zeros((RPT, D), jnp.float32)

    deg = _deg_kernel()(dst_p, zeros1, ones_k)
    y1 = _mm_scale(deg, x_p, W1)
    a1 = _edge_kernel()(y1, sd_p, zrows)
    y2 = _mid(deg, a1, y1, b1r, W2)
    a2 = _edge_kernel()(y2, sd_p, zrows)
    out = _fin(deg, a2, y2, b2r)
    return out[:N]


# speedup vs baseline: 1.0534x; 1.0534x over previous
import functools

import jax
import jax.numpy as jnp
import numpy as np
from jax import lax
from jax.experimental import pallas as pl
from jax.experimental.pallas import tpu as pltpu
from jax.experimental.pallas import tpu_sc as plsc

N = 10000
D = 128
E = 320000
NP = 10240
NW = 32
K = 120
NBLK = 84
EPW = K * NBLK
EP = NW * EPW
RPT = NP // 16
NBUF = 3
NSLOT = 4

def _mesh():
    return plsc.VectorSubcoreMesh(
        core_axis_name="c", subcore_axis_name="s", num_cores=2,
        num_subcores=16)


def _deg_body(dst_hbm, zeros1_hbm, ones_hbm, out_hbm, didx, ones_v, deg_sp):
    c = lax.axis_index("c")
    s = lax.axis_index("s")
    w = s * 2 + c
    pltpu.sync_copy(dst_hbm.at[w], didx)
    pltpu.sync_copy(ones_hbm, ones_v)
    pltpu.sync_copy(zeros1_hbm, deg_sp.at[pl.ds(s * RPT, RPT)])
    plsc.subcore_barrier()

    def body(j, carry):
        pltpu.sync_copy(ones_v, deg_sp.at[didx.at[j]], add=True)
        return carry

    lax.fori_loop(0, NBLK, body, 0)
    plsc.subcore_barrier()
    pltpu.sync_copy(deg_sp.at[pl.ds(s * RPT, RPT)],
                    out_hbm.at[c, pl.ds(s * RPT, RPT)])


@functools.lru_cache(maxsize=None)
def _deg_kernel():
    return pl.kernel(
        _deg_body,
        out_type=jax.ShapeDtypeStruct((2, NP), jnp.float32),
        mesh=_mesh(),
        scratch_types=[
            pltpu.VMEM((NBLK, K), jnp.int32),
            pltpu.VMEM((K,), jnp.float32),
            pltpu.VMEM_SHARED((NP,), jnp.float32),
        ],
    )


def _edge_body(y_hbm, sd_hbm, zrows_hbm, out_hbm,
               ring, bufs, acc_sp,
               isem0, isem1, isem2, isem3,
               gsem0, gsem1, gsem2, ssem0, ssem1, ssem2):
    c = lax.axis_index("c")
    s = lax.axis_index("s")
    w = s * 2 + c
    isems = (isem0, isem1, isem2, isem3)
    gsems = (gsem0, gsem1, gsem2)
    ssems = (ssem0, ssem1, ssem2)

    def iload(blk, slot):
        return pltpu.async_copy(
            sd_hbm.at[w, blk], ring.at[pl.ds(2 * slot, 2)], isems[slot])

    def iwait(slot):
        pltpu.make_async_copy(sd_hbm.at[w, 0], ring.at[pl.ds(2 * slot, 2)],
                              isems[slot]).wait()

    def gather(blk, slot, buf):
        return pltpu.async_copy(
            y_hbm.at[ring.at[2 * slot]], bufs.at[buf], gsems[buf])

    def gwait(slot, buf):
        pltpu.make_async_copy(y_hbm.at[ring.at[2 * slot]], bufs.at[buf],
                              gsems[buf]).wait()

    def scat(slot, buf):
        return pltpu.async_copy(
            bufs.at[buf], acc_sp.at[ring.at[2 * slot + 1]], ssems[buf],
            add=True)

    def swait(slot, buf):
        pltpu.make_async_copy(bufs.at[buf], acc_sp.at[ring.at[2 * slot + 1]],
                              ssems[buf]).wait()

    pltpu.sync_copy(zrows_hbm, acc_sp.at[pl.ds(s * RPT, RPT)])
    plsc.subcore_barrier()

    for r in range(NSLOT):
        iload(r, r)
    iwait(0)
    gather(0, 0, 0)
    iwait(1)
    gather(1, 1, 1)

    def body(j, carry):
        for b in range(12):
            jb = j * 12 + b
            sl = b % NSLOT
            bf = b % NBUF
            gwait(sl, bf)
            scat(sl, bf)

            @pl.when(jb + 2 < NBLK)
            def _next_gather():
                @pl.when(jb >= 1)
                def _wait_prev_scat():
                    swait((sl - 1) % NSLOT, (bf - 1) % NBUF)
                iwait((sl + 2) % NSLOT)
                gather(jb + 2, (sl + 2) % NSLOT, (bf + 2) % NBUF)

                @pl.when(jb >= 1)
                def _next_iload():
                    @pl.when(jb + 3 < NBLK)
                    def _do_iload():
                        iload(jb + 3, (sl + 3) % NSLOT)
        return carry

    lax.fori_loop(0, NBLK // 12, body, 0)
    for t in range(3):
        blk = NBLK - 3 + t
        swait(blk % NSLOT, blk % NBUF)
    plsc.subcore_barrier()
    pltpu.sync_copy(acc_sp.at[pl.ds(s * RPT, RPT)],
                    out_hbm.at[c, pl.ds(s * RPT, RPT)])


@functools.lru_cache(maxsize=None)
def _edge_kernel():
    return pl.kernel(
        _edge_body,
        out_type=jax.ShapeDtypeStruct((2, NP, D), jnp.float32),
        mesh=_mesh(),
        scratch_types=[
            pltpu.VMEM((2 * NSLOT, K), jnp.int32),
            pltpu.VMEM((NBUF, K, D), jnp.float32),
            pltpu.VMEM_SHARED((NP, D), jnp.float32),
        ] + [pltpu.SemaphoreType.DMA] * 10,
    )


_BM = 1024
_GRID = NP // _BM


def _dinv_col(deg_ref):
    deg = deg_ref[...]
    ones = jnp.ones((2, 1), jnp.float32)
    degc = lax.dot_general(deg, ones, (((0,), (0,)), ((), ())))
    return 1.0 / jnp.sqrt(degc + 1.0)


def _mm_scale_body(deg_ref, x_ref, w_ref, o_ref):
    dinv = _dinv_col(deg_ref)
    xw = jnp.dot(x_ref[...], w_ref[...], preferred_element_type=jnp.float32)
    o_ref[...] = xw * dinv


_mm_scale = pl.pallas_call(
    _mm_scale_body,
    grid=(_GRID,),
    in_specs=[
        pl.BlockSpec((2, _BM), lambda i: (0, i)),
        pl.BlockSpec((_BM, D), lambda i: (i, 0)),
        pl.BlockSpec((D, D), lambda i: (0, 0)),
    ],
    out_specs=pl.BlockSpec((_BM, D), lambda i: (i, 0)),
    out_shape=jax.ShapeDtypeStruct((NP, D), jnp.float32),
)


def _mid_body(deg_ref, a_ref, y_ref, b_ref, w_ref, o_ref):
    dinv = _dinv_col(deg_ref)
    h = (a_ref[0] + a_ref[1] + y_ref[...]) * dinv + b_ref[...]
    o_ref[...] = jnp.dot(h, w_ref[...],
                         preferred_element_type=jnp.float32) * dinv


_mid = pl.pallas_call(
    _mid_body,
    grid=(_GRID,),
    in_specs=[
        pl.BlockSpec((2, _BM), lambda i: (0, i)),
        pl.BlockSpec((2, _BM, D), lambda i: (0, i, 0)),
        pl.BlockSpec((_BM, D), lambda i: (i, 0)),
        pl.BlockSpec((1, D), lambda i: (0, 0)),
        pl.BlockSpec((D, D), lambda i: (0, 0)),
    ],
    out_specs=pl.BlockSpec((_BM, D), lambda i: (i, 0)),
    out_shape=jax.ShapeDtypeStruct((NP, D), jnp.float32),
)


def _fin_body(deg_ref, a_ref, y_ref, b_ref, o_ref):
    deg = deg_ref[...][:, :, 0]
    ones = jnp.ones((2, 1), jnp.float32)
    degc = lax.dot_general(deg, ones, (((0,), (0,)), ((), ())))
    dinv = 1.0 / jnp.sqrt(degc + 1.0)
    o_ref[...] = (a_ref[0] + a_ref[1] + y_ref[...]) * dinv + b_ref[...]


_BF = 1000

_fin = pl.pallas_call(
    _fin_body,
    grid=(N // _BF,),
    in_specs=[
        pl.BlockSpec((2, _BF, 1), lambda i: (0, i, 0)),
        pl.BlockSpec((2, _BF, D), lambda i: (0, i, 0)),
        pl.BlockSpec((_BF, D), lambda i: (i, 0)),
        pl.BlockSpec((1, D), lambda i: (0, 0)),
    ],
    out_specs=pl.BlockSpec((_BF, D), lambda i: (i, 0)),
    out_shape=jax.ShapeDtypeStruct((N, D), jnp.float32),
)


def kernel(x, edge_index, W1, b1, W2, b2):
    src, dst = edge_index[0], edge_index[1]
    npad = EP - E
    pad_src = jnp.asarray(np.arange(npad, dtype=np.int32) % N)
    pad_dst = jnp.asarray(N + (np.arange(npad, dtype=np.int32) % (NP - N)))
    src_p = jnp.concatenate([src, pad_src]).reshape(NW, NBLK, K)
    dst_p = jnp.concatenate([dst, pad_dst]).reshape(NW, NBLK, K)
    sd_p = jnp.stack([src_p, dst_p], axis=2)

    x_p = jnp.pad(x, ((0, NP - N), (0, 0)))
    b1r = b1.reshape(1, D)
    b2r = b2.reshape(1, D)

    zeros1 = jnp.zeros((RPT,), jnp.float32)
    ones_k = jnp.ones((K,), jnp.float32)
    zrows = jnp.zeros((RPT, D), jnp.float32)

    deg = _deg_kernel()(dst_p, zeros1, ones_k)
    y1 = _mm_scale(deg, x_p, W1)
    a1 = _edge_kernel()(y1, sd_p, zrows)
    y2 = _mid(deg, a1, y1, b1r, W2)
    a2 = _edge_kernel()(y2, sd_p, zrows)
    return _fin(deg.reshape(2, NP, 1), a2, y2, b2r)


# speedup vs baseline: 1.0550x; 1.0015x over previous
import functools

import jax
import jax.numpy as jnp
import numpy as np
from jax import lax
from jax.experimental import pallas as pl
from jax.experimental.pallas import tpu as pltpu
from jax.experimental.pallas import tpu_sc as plsc

N = 10000
D = 128
E = 320000
NP = 10240
NW = 32
K = 120
NBLK = 84
EPW = K * NBLK
EP = NW * EPW
RPT = NP // 16
NBUF = 3
NSLOT = 4

def _mesh():
    return plsc.VectorSubcoreMesh(
        core_axis_name="c", subcore_axis_name="s", num_cores=2,
        num_subcores=16)


def _deg_body(dst_hbm, zeros1_hbm, ones_hbm, out_hbm, didx, ones_v, deg_sp):
    c = lax.axis_index("c")
    s = lax.axis_index("s")
    w = s * 2 + c
    pltpu.sync_copy(dst_hbm.at[w], didx)
    pltpu.sync_copy(ones_hbm, ones_v)
    pltpu.sync_copy(zeros1_hbm, deg_sp.at[pl.ds(s * RPT, RPT)])
    plsc.subcore_barrier()

    def body(j, carry):
        pltpu.sync_copy(ones_v, deg_sp.at[didx.at[j]], add=True)
        return carry

    lax.fori_loop(0, NBLK, body, 0)
    plsc.subcore_barrier()
    pltpu.sync_copy(deg_sp.at[pl.ds(s * RPT, RPT)],
                    out_hbm.at[c, pl.ds(s * RPT, RPT)])


@functools.lru_cache(maxsize=None)
def _deg_kernel():
    return pl.kernel(
        _deg_body,
        out_type=jax.ShapeDtypeStruct((2, NP), jnp.float32),
        mesh=_mesh(),
        scratch_types=[
            pltpu.VMEM((NBLK, K), jnp.int32),
            pltpu.VMEM((K,), jnp.float32),
            pltpu.VMEM_SHARED((NP,), jnp.float32),
        ],
    )


def _edge_body(y_hbm, src_hbm, dst_hbm, zrows_hbm, out_hbm,
               ring, bufs, acc_sp,
               isem0, isem1, isem2, isem3,
               gsem0, gsem1, gsem2, ssem0, ssem1, ssem2):
    c = lax.axis_index("c")
    s = lax.axis_index("s")
    w = s * 2 + c
    isems = (isem0, isem1, isem2, isem3)
    gsems = (gsem0, gsem1, gsem2)
    ssems = (ssem0, ssem1, ssem2)

    def iload(blk, slot):
        pltpu.async_copy(src_hbm.at[w, blk],
                         ring.at[pl.ds(2 * slot, 1)], isems[slot])
        pltpu.async_copy(dst_hbm.at[w, blk],
                         ring.at[pl.ds(2 * slot + 1, 1)], isems[slot])

    def iwait(slot):
        pltpu.make_async_copy(src_hbm.at[w, 0],
                              ring.at[pl.ds(2 * slot, 1)], isems[slot]).wait()
        pltpu.make_async_copy(src_hbm.at[w, 0],
                              ring.at[pl.ds(2 * slot + 1, 1)],
                              isems[slot]).wait()

    def gather(blk, slot, buf):
        return pltpu.async_copy(
            y_hbm.at[ring.at[2 * slot]], bufs.at[buf], gsems[buf])

    def gwait(slot, buf):
        pltpu.make_async_copy(y_hbm.at[ring.at[2 * slot]], bufs.at[buf],
                              gsems[buf]).wait()

    def scat(slot, buf):
        return pltpu.async_copy(
            bufs.at[buf], acc_sp.at[ring.at[2 * slot + 1]], ssems[buf],
            add=True)

    def swait(slot, buf):
        pltpu.make_async_copy(bufs.at[buf], acc_sp.at[ring.at[2 * slot + 1]],
                              ssems[buf]).wait()

    pltpu.sync_copy(zrows_hbm, acc_sp.at[pl.ds(s * RPT, RPT)])
    plsc.subcore_barrier()

    for r in range(NSLOT):
        iload(r, r)
    iwait(0)
    gather(0, 0, 0)
    iwait(1)
    gather(1, 1, 1)

    def body(j, carry):
        for b in range(12):
            jb = j * 12 + b
            sl = b % NSLOT
            bf = b % NBUF
            gwait(sl, bf)
            scat(sl, bf)

            @pl.when(jb + 2 < NBLK)
            def _next_gather():
                @pl.when(jb >= 1)
                def _wait_prev_scat():
                    swait((sl - 1) % NSLOT, (bf - 1) % NBUF)
                iwait((sl + 2) % NSLOT)
                gather(jb + 2, (sl + 2) % NSLOT, (bf + 2) % NBUF)

                @pl.when(jb >= 1)
                def _next_iload():
                    @pl.when(jb + 3 < NBLK)
                    def _do_iload():
                        iload(jb + 3, (sl + 3) % NSLOT)
        return carry

    lax.fori_loop(0, NBLK // 12, body, 0)
    for t in range(3):
        blk = NBLK - 3 + t
        swait(blk % NSLOT, blk % NBUF)
    plsc.subcore_barrier()
    pltpu.sync_copy(acc_sp.at[pl.ds(s * RPT, RPT)],
                    out_hbm.at[c, pl.ds(s * RPT, RPT)])


@functools.lru_cache(maxsize=None)
def _edge_kernel():
    return pl.kernel(
        _edge_body,
        out_type=jax.ShapeDtypeStruct((2, NP, D), jnp.float32),
        mesh=_mesh(),
        scratch_types=[
            pltpu.VMEM((2 * NSLOT, K), jnp.int32),
            pltpu.VMEM((NBUF, K, D), jnp.float32),
            pltpu.VMEM_SHARED((NP, D), jnp.float32),
        ] + [pltpu.SemaphoreType.DMA] * 10,
    )


_BM = 1024
_GRID = NP // _BM


def _dinv_col(deg_ref):
    deg = deg_ref[...]
    ones = jnp.ones((2, 1), jnp.float32)
    degc = lax.dot_general(deg, ones, (((0,), (0,)), ((), ())))
    return 1.0 / jnp.sqrt(degc + 1.0)


def _mm_scale_body(deg_ref, x_ref, w_ref, o_ref):
    dinv = _dinv_col(deg_ref)
    xw = jnp.dot(x_ref[...], w_ref[...], preferred_element_type=jnp.float32)
    o_ref[...] = xw * dinv


_mm_scale = pl.pallas_call(
    _mm_scale_body,
    grid=(_GRID,),
    in_specs=[
        pl.BlockSpec((2, _BM), lambda i: (0, i)),
        pl.BlockSpec((_BM, D), lambda i: (i, 0)),
        pl.BlockSpec((D, D), lambda i: (0, 0)),
    ],
    out_specs=pl.BlockSpec((_BM, D), lambda i: (i, 0)),
    out_shape=jax.ShapeDtypeStruct((NP, D), jnp.float32),
)


def _mid_body(deg_ref, a_ref, y_ref, b_ref, w_ref, o_ref):
    dinv = _dinv_col(deg_ref)
    h = (a_ref[0] + a_ref[1] + y_ref[...]) * dinv + b_ref[...]
    o_ref[...] = jnp.dot(h, w_ref[...],
                         preferred_element_type=jnp.float32) * dinv


_mid = pl.pallas_call(
    _mid_body,
    grid=(_GRID,),
    in_specs=[
        pl.BlockSpec((2, _BM), lambda i: (0, i)),
        pl.BlockSpec((2, _BM, D), lambda i: (0, i, 0)),
        pl.BlockSpec((_BM, D), lambda i: (i, 0)),
        pl.BlockSpec((1, D), lambda i: (0, 0)),
        pl.BlockSpec((D, D), lambda i: (0, 0)),
    ],
    out_specs=pl.BlockSpec((_BM, D), lambda i: (i, 0)),
    out_shape=jax.ShapeDtypeStruct((NP, D), jnp.float32),
)


def _fin_body(deg_ref, a_ref, y_ref, b_ref, o_ref):
    deg = deg_ref[...][:, :, 0]
    ones = jnp.ones((2, 1), jnp.float32)
    degc = lax.dot_general(deg, ones, (((0,), (0,)), ((), ())))
    dinv = 1.0 / jnp.sqrt(degc + 1.0)
    o_ref[...] = (a_ref[0] + a_ref[1] + y_ref[...]) * dinv + b_ref[...]


_BF = 1000

_fin = pl.pallas_call(
    _fin_body,
    grid=(N // _BF,),
    in_specs=[
        pl.BlockSpec((2, _BF, 1), lambda i: (0, i, 0)),
        pl.BlockSpec((2, _BF, D), lambda i: (0, i, 0)),
        pl.BlockSpec((_BF, D), lambda i: (i, 0)),
        pl.BlockSpec((1, D), lambda i: (0, 0)),
    ],
    out_specs=pl.BlockSpec((_BF, D), lambda i: (i, 0)),
    out_shape=jax.ShapeDtypeStruct((N, D), jnp.float32),
)


def kernel(x, edge_index, W1, b1, W2, b2):
    src, dst = edge_index[0], edge_index[1]
    npad = EP - E
    pad_src = jnp.asarray(np.arange(npad, dtype=np.int32) % N)
    pad_dst = jnp.asarray(N + (np.arange(npad, dtype=np.int32) % (NP - N)))
    src_p = jnp.concatenate([src, pad_src]).reshape(NW, NBLK, 1, K)
    dst_p = jnp.concatenate([dst, pad_dst]).reshape(NW, NBLK, 1, K)
    dst_b = dst_p.reshape(NW, NBLK, K)

    x_p = jnp.pad(x, ((0, NP - N), (0, 0)))
    b1r = b1.reshape(1, D)
    b2r = b2.reshape(1, D)

    zeros1 = jnp.zeros((RPT,), jnp.float32)
    ones_k = jnp.ones((K,), jnp.float32)
    zrows = jnp.zeros((RPT, D), jnp.float32)

    deg = _deg_kernel()(dst_b, zeros1, ones_k)
    y1 = _mm_scale(deg, x_p, W1)
    a1 = _edge_kernel()(y1, src_p, dst_p, zrows)
    y2 = _mid(deg, a1, y1, b1r, W2)
    a2 = _edge_kernel()(y2, src_p, dst_p, zrows)
    return _fin(deg.reshape(2, NP, 1), a2, y2, b2r)


# speedup vs baseline: 1.0781x; 1.0219x over previous
import functools

import jax
import jax.numpy as jnp
import numpy as np
from jax import lax
from jax.experimental import pallas as pl
from jax.experimental.pallas import tpu as pltpu
from jax.experimental.pallas import tpu_sc as plsc

N = 10000
D = 128
E = 320000
NP = 10240
NW = 32
K = 120
NBLK = 84
EPW = K * NBLK
EP = NW * EPW
RPT = NP // 16
NBUF = 3
NSLOT = 4

def _mesh():
    return plsc.VectorSubcoreMesh(
        core_axis_name="c", subcore_axis_name="s", num_cores=2,
        num_subcores=16)


def _deg_body(dst_hbm, zeros1_hbm, ones_hbm, out_hbm, didx, ones_v, deg_sp):
    c = lax.axis_index("c")
    s = lax.axis_index("s")
    w = s * 2 + c
    pltpu.sync_copy(dst_hbm.at[w], didx)
    pltpu.sync_copy(ones_hbm, ones_v)
    pltpu.sync_copy(zeros1_hbm, deg_sp.at[pl.ds(s * RPT, RPT)])
    plsc.subcore_barrier()

    def body(j, carry):
        pltpu.sync_copy(ones_v, deg_sp.at[didx.at[j]], add=True)
        return carry

    lax.fori_loop(0, NBLK, body, 0)
    plsc.subcore_barrier()
    pltpu.sync_copy(deg_sp.at[pl.ds(s * RPT, RPT)],
                    out_hbm.at[c, pl.ds(s * RPT, RPT)])


@functools.lru_cache(maxsize=None)
def _deg_kernel():
    return pl.kernel(
        _deg_body,
        out_type=jax.ShapeDtypeStruct((2, NP), jnp.float32),
        mesh=_mesh(),
        scratch_types=[
            pltpu.VMEM((NBLK, K), jnp.int32),
            pltpu.VMEM((K,), jnp.float32),
            pltpu.VMEM_SHARED((NP,), jnp.float32),
        ],
    )


def _edge_body(y_hbm, src_hbm, dst_hbm, zrows_hbm, out_hbm,
               ring, bufs, acc_sp,
               isem0, isem1, isem2, isem3,
               gsem0, gsem1, gsem2, ssem0, ssem1, ssem2):
    c = lax.axis_index("c")
    s = lax.axis_index("s")
    w = s * 2 + c
    isems = (isem0, isem1, isem2, isem3)
    gsems = (gsem0, gsem1, gsem2)
    ssems = (ssem0, ssem1, ssem2)

    def iload(blk, slot):
        pltpu.async_copy(src_hbm.at[w, blk],
                         ring.at[pl.ds(2 * slot, 1)], isems[slot])
        pltpu.async_copy(dst_hbm.at[w, blk],
                         ring.at[pl.ds(2 * slot + 1, 1)], isems[slot])

    def iwait(slot):
        pltpu.make_async_copy(src_hbm.at[w, 0],
                              ring.at[pl.ds(2 * slot, 1)], isems[slot]).wait()
        pltpu.make_async_copy(src_hbm.at[w, 0],
                              ring.at[pl.ds(2 * slot + 1, 1)],
                              isems[slot]).wait()

    def gather(blk, slot, buf):
        return pltpu.async_copy(
            y_hbm.at[ring.at[2 * slot]], bufs.at[buf], gsems[buf])

    def gwait(slot, buf):
        pltpu.make_async_copy(y_hbm.at[ring.at[2 * slot]], bufs.at[buf],
                              gsems[buf]).wait()

    def scat(slot, buf):
        return pltpu.async_copy(
            bufs.at[buf], acc_sp.at[ring.at[2 * slot + 1]], ssems[buf],
            add=True)

    def swait(slot, buf):
        pltpu.make_async_copy(bufs.at[buf], acc_sp.at[ring.at[2 * slot + 1]],
                              ssems[buf]).wait()

    pltpu.sync_copy(zrows_hbm, acc_sp.at[pl.ds(s * RPT, RPT)])
    plsc.subcore_barrier()

    for r in range(NSLOT):
        iload(r, r)
    iwait(0)
    gather(0, 0, 0)
    iwait(1)
    gather(1, 1, 1)

    def body(j, carry):
        for b in range(12):
            jb = j * 12 + b
            sl = b % NSLOT
            bf = b % NBUF
            gwait(sl, bf)
            scat(sl, bf)

            @pl.when(jb + 2 < NBLK)
            def _next_gather():
                @pl.when(jb >= 1)
                def _wait_prev_scat():
                    swait((sl - 1) % NSLOT, (bf - 1) % NBUF)
                iwait((sl + 2) % NSLOT)
                gather(jb + 2, (sl + 2) % NSLOT, (bf + 2) % NBUF)

                @pl.when(jb >= 1)
                def _next_iload():
                    @pl.when(jb + 3 < NBLK)
                    def _do_iload():
                        iload(jb + 3, (sl + 3) % NSLOT)
        return carry

    lax.fori_loop(0, NBLK // 12, body, 0)
    for t in range(3):
        blk = NBLK - 3 + t
        swait(blk % NSLOT, blk % NBUF)
    plsc.subcore_barrier()
    pltpu.sync_copy(acc_sp.at[pl.ds(s * RPT, RPT)],
                    out_hbm.at[c, pl.ds(s * RPT, RPT)])


@functools.lru_cache(maxsize=None)
def _edge_kernel():
    return pl.kernel(
        _edge_body,
        out_type=jax.ShapeDtypeStruct((2, NP, D), jnp.float32),
        mesh=_mesh(),
        scratch_types=[
            pltpu.VMEM((2 * NSLOT, K), jnp.int32),
            pltpu.VMEM((NBUF, K, D), jnp.float32),
            pltpu.VMEM_SHARED((NP, D), jnp.float32),
        ] + [pltpu.SemaphoreType.DMA] * 10,
    )


_BM = 2048
_GRID = NP // _BM


def _dinv_col(deg_ref):
    deg = deg_ref[...]
    ones = jnp.ones((2, 1), jnp.float32)
    degc = lax.dot_general(deg, ones, (((0,), (0,)), ((), ())))
    return 1.0 / jnp.sqrt(degc + 1.0)


def _mm_scale_body(deg_ref, x_ref, w_ref, o_ref):
    dinv = _dinv_col(deg_ref)
    xw = jnp.dot(x_ref[...], w_ref[...], preferred_element_type=jnp.float32)
    o_ref[...] = xw * dinv


_mm_scale = pl.pallas_call(
    _mm_scale_body,
    grid=(_GRID,),
    in_specs=[
        pl.BlockSpec((2, _BM), lambda i: (0, i)),
        pl.BlockSpec((_BM, D), lambda i: (i, 0)),
        pl.BlockSpec((D, D), lambda i: (0, 0)),
    ],
    out_specs=pl.BlockSpec((_BM, D), lambda i: (i, 0)),
    out_shape=jax.ShapeDtypeStruct((NP, D), jnp.float32),
)


def _mid_body(deg_ref, a_ref, y_ref, b_ref, w_ref, o_ref):
    dinv = _dinv_col(deg_ref)
    h = (a_ref[0] + a_ref[1] + y_ref[...]) * dinv + b_ref[...]
    o_ref[...] = jnp.dot(h, w_ref[...],
                         preferred_element_type=jnp.float32) * dinv


_mid = pl.pallas_call(
    _mid_body,
    grid=(_GRID,),
    in_specs=[
        pl.BlockSpec((2, _BM), lambda i: (0, i)),
        pl.BlockSpec((2, _BM, D), lambda i: (0, i, 0)),
        pl.BlockSpec((_BM, D), lambda i: (i, 0)),
        pl.BlockSpec((1, D), lambda i: (0, 0)),
        pl.BlockSpec((D, D), lambda i: (0, 0)),
    ],
    out_specs=pl.BlockSpec((_BM, D), lambda i: (i, 0)),
    out_shape=jax.ShapeDtypeStruct((NP, D), jnp.float32),
)


def _fin_body(deg_ref, a_ref, y_ref, b_ref, o_ref):
    deg = deg_ref[...][:, :, 0]
    ones = jnp.ones((2, 1), jnp.float32)
    degc = lax.dot_general(deg, ones, (((0,), (0,)), ((), ())))
    dinv = 1.0 / jnp.sqrt(degc + 1.0)
    o_ref[...] = (a_ref[0] + a_ref[1] + y_ref[...]) * dinv + b_ref[...]


_BF = 1000

_fin = pl.pallas_call(
    _fin_body,
    grid=(N // _BF,),
    in_specs=[
        pl.BlockSpec((2, _BF, 1), lambda i: (0, i, 0)),
        pl.BlockSpec((2, _BF, D), lambda i: (0, i, 0)),
        pl.BlockSpec((_BF, D), lambda i: (i, 0)),
        pl.BlockSpec((1, D), lambda i: (0, 0)),
    ],
    out_specs=pl.BlockSpec((_BF, D), lambda i: (i, 0)),
    out_shape=jax.ShapeDtypeStruct((N, D), jnp.float32),
)


def kernel(x, edge_index, W1, b1, W2, b2):
    src, dst = edge_index[0], edge_index[1]
    npad = EP - E
    pad_src = jnp.asarray(np.arange(npad, dtype=np.int32) % N)
    pad_dst = jnp.asarray(N + (np.arange(npad, dtype=np.int32) % (NP - N)))
    src_p = jnp.concatenate([src, pad_src]).reshape(NW, NBLK, 1, K)
    dst_p = jnp.concatenate([dst, pad_dst]).reshape(NW, NBLK, 1, K)
    dst_b = dst_p.reshape(NW, NBLK, K)

    x_p = jnp.pad(x, ((0, NP - N), (0, 0)))
    b1r = b1.reshape(1, D)
    b2r = b2.reshape(1, D)

    zeros1 = jnp.zeros((RPT,), jnp.float32)
    ones_k = jnp.ones((K,), jnp.float32)
    zrows = jnp.zeros((RPT, D), jnp.float32)

    deg = _deg_kernel()(dst_b, zeros1, ones_k)
    y1 = _mm_scale(deg, x_p, W1)
    a1 = _edge_kernel()(y1, src_p, dst_p, zrows)
    y2 = _mid(deg, a1, y1, b1r, W2)
    a2 = _edge_kernel()(y2, src_p, dst_p, zrows)
    return _fin(deg.reshape(2, NP, 1), a2, y2, b2r)
